# Initial kernel scaffold; baseline (speedup 1.0000x reference)
import jax
import jax.numpy as jnp
from jax.experimental import pallas as pl


def kernel(test, question, tag, testTag, num_feat, E_test, E_q, E_tag, E_tt, W_cat, b_cat, g_cat, bt_cat, W_num, b_num, g_num, bt_num, g_out, bt_out):
    raise NotImplementedError("write your pallas kernel here")


# speedup vs baseline: 2.4043x; 2.4043x over previous
import functools

import jax
import jax.numpy as jnp
from jax import lax
from jax.experimental import pallas as pl
from jax.experimental.pallas import tpu as pltpu
from jax.experimental.pallas import tpu_sc as plsc

B, L = 4096, 200
NTOK = B * L
INTD = 21
D = 32
HID = 64
EPS = 1e-6

NC, NS = 2, 16
NW = NC * NS
TOK_PER_W = NTOK // NW
CH = 512
CHB = CH // 128
NCHUNK = TOK_PER_W // CH


def _proj_body(e1, e2, e3, e4, w1, w2, w3, w4, p1, p2, p3, p4):
    p1[...] = jnp.dot(e1[...], w1[...], preferred_element_type=jnp.float32)
    p2[...] = jnp.dot(e2[...], w2[...], preferred_element_type=jnp.float32)
    p3[...] = jnp.dot(e3[...], w3[...], preferred_element_type=jnp.float32)
    p4[...] = jnp.dot(e4[...], w4[...], preferred_element_type=jnp.float32)


def _project_tables(E1, E2, E3, E4, W_cat):
    w1, w2, w3, w4 = (W_cat[0:21], W_cat[21:42], W_cat[42:63], W_cat[63:84])
    out_shapes = tuple(
        jax.ShapeDtypeStruct((e.shape[0], D), jnp.float32)
        for e in (E1, E2, E3, E4)
    )
    return pl.pallas_call(
        _proj_body,
        out_shape=out_shapes,
    )(E1, E2, E3, E4, w1, w2, w3, w4)


def _sc_body(i1, i2, i3, i4, p1, p2, p3, p4, out,
             i1v, i2v, i3v, i4v, r1, r2, r3, r4, sv, sem):
    wid = lax.axis_index("s") * NC + lax.axis_index("c")

    def chunk_body(c, _):
        base = wid * TOK_PER_W + c * CH
        brow = base // 128

        pltpu.sync_copy(i1.at[pl.ds(brow, CHB)], i1v)
        pltpu.sync_copy(i2.at[pl.ds(brow, CHB)], i2v)
        pltpu.sync_copy(i3.at[pl.ds(brow, CHB)], i3v)
        pltpu.sync_copy(i4.at[pl.ds(brow, CHB)], i4v)

        descs = []
        for j in range(CHB):
            dst = pl.ds(j * 128, 128)
            descs.append(pltpu.async_copy(p1.at[i1v.at[j]], r1.at[dst], sem))
            descs.append(pltpu.async_copy(p2.at[i2v.at[j]], r2.at[dst], sem))
            descs.append(pltpu.async_copy(p3.at[i3v.at[j]], r3.at[dst], sem))
            descs.append(pltpu.async_copy(p4.at[i4v.at[j]], r4.at[dst], sem))
        for d in descs:
            d.wait()

        UNROLL = 8

        def sum_body(t, _):
            for u in range(UNROLL):
                row = t * UNROLL + u
                for k in range(2):
                    sl = pl.ds(k * 16, 16)
                    sv[row, sl] = (r1[row, sl] + r2[row, sl]
                                   + r3[row, sl] + r4[row, sl])
            return _

        lax.fori_loop(0, CH // UNROLL, sum_body, None)
        pltpu.sync_copy(sv, out.at[pl.ds(base, CH)])
        return _

    lax.fori_loop(0, NCHUNK, chunk_body, None)


def _sc_gather_sum(idx1, idx2, idx3, idx4, P1, P2, P3, P4):
    mesh = plsc.VectorSubcoreMesh(core_axis_name="c", subcore_axis_name="s")
    fn = functools.partial(
        pl.kernel,
        out_type=jax.ShapeDtypeStruct((NTOK, D), jnp.float32),
        mesh=mesh,
        scratch_types=[
            pltpu.VMEM((CHB, 128), jnp.int32),
            pltpu.VMEM((CHB, 128), jnp.int32),
            pltpu.VMEM((CHB, 128), jnp.int32),
            pltpu.VMEM((CHB, 128), jnp.int32),
            pltpu.VMEM((CH, D), jnp.float32),
            pltpu.VMEM((CH, D), jnp.float32),
            pltpu.VMEM((CH, D), jnp.float32),
            pltpu.VMEM((CH, D), jnp.float32),
            pltpu.VMEM((CH, D), jnp.float32),
            pltpu.SemaphoreType.DMA,
        ],
        compiler_params=pltpu.CompilerParams(use_tc_tiling_on_sc=False),
    )(_sc_body)
    return fn(idx1, idx2, idx3, idx4, P1, P2, P3, P4)


def _ln_body(s_ref, x_ref, bcat, gcat, btcat, wnum, bnum, gnum, btnum,
             gout, btout, o_ref):
    s = s_ref[...] + bcat[...]
    mu = jnp.mean(s, axis=-1, keepdims=True)
    xc = s - mu
    var = jnp.mean(xc * xc, axis=-1, keepdims=True)
    cat = xc * lax.rsqrt(var + EPS) * gcat[...] + btcat[...]

    x = x_ref[...]
    h = x * wnum[...] + bnum[...]
    mu2 = jnp.mean(h, axis=-1, keepdims=True)
    hc = h - mu2
    var2 = jnp.mean(hc * hc, axis=-1, keepdims=True)
    num = hc * lax.rsqrt(var2 + EPS) * gnum[...] + btnum[...]

    o = jnp.concatenate([cat, num], axis=-1)
    mu3 = jnp.mean(o, axis=-1, keepdims=True)
    oc = o - mu3
    var3 = jnp.mean(oc * oc, axis=-1, keepdims=True)
    o_ref[...] = oc * lax.rsqrt(var3 + EPS) * gout[...] + btout[...]


def _ln_stage(s, x, b_cat, g_cat, bt_cat, W_num, b_num, g_num, bt_num,
              g_out, bt_out):
    TBLK = 2048
    grid = (NTOK // TBLK,)
    row = lambda i: (i, 0)
    const = lambda i: (0, 0)
    vec32 = pl.BlockSpec((1, D), const)
    vec64 = pl.BlockSpec((1, HID), const)
    return pl.pallas_call(
        _ln_body,
        grid=grid,
        in_specs=[
            pl.BlockSpec((TBLK, D), row),
            pl.BlockSpec((TBLK, 1), row),
            vec32, vec32, vec32, vec32, vec32, vec32, vec32,
            vec64, vec64,
        ],
        out_specs=pl.BlockSpec((TBLK, HID), row),
        out_shape=jax.ShapeDtypeStruct((NTOK, HID), jnp.float32),
    )(s, x,
      b_cat.reshape(1, D), g_cat.reshape(1, D), bt_cat.reshape(1, D),
      W_num.reshape(1, D), b_num.reshape(1, D), g_num.reshape(1, D),
      bt_num.reshape(1, D), g_out.reshape(1, HID), bt_out.reshape(1, HID))


def kernel(test, question, tag, testTag, num_feat,
           E_test, E_q, E_tag, E_tt,
           W_cat, b_cat, g_cat, bt_cat,
           W_num, b_num, g_num, bt_num,
           g_out, bt_out):
    P1, P2, P3, P4 = _project_tables(E_test, E_q, E_tag, E_tt, W_cat)
    idx = lambda a: a.reshape(NTOK // 128, 128)
    s = _sc_gather_sum(idx(test), idx(question), idx(tag), idx(testTag),
                       P1, P2, P3, P4)
    out = _ln_stage(s, num_feat.reshape(NTOK, 1),
                    b_cat, g_cat, bt_cat, W_num, b_num, g_num, bt_num,
                    g_out, bt_out)
    return out.reshape(B, L, HID)
